# baseline (speedup 1.0000x reference)
import functools
import jax
import jax.numpy as jnp
from jax import lax
from jax.experimental import pallas as pl
from jax.experimental.pallas import tpu as pltpu
from jax.experimental.pallas import tpu_sc as plsc

_NUM_SC_CORES = 2
_NUM_SC_SUBCORES = 16
_NUM_SC_WORKERS = _NUM_SC_CORES * _NUM_SC_SUBCORES


def _topk_body(K, BK, x_ref, xn_ref, mem_ref, mnorm_ref,
               score_ref, idx_ref, qn_ref, tmp_ref):
    k = pl.program_id(0)

    @pl.when(k == 0)
    def _():
        qn_ref[...] = x_ref[...] / jnp.maximum(xn_ref[...], 1e-12)

    mn = mem_ref[...] / jnp.maximum(mnorm_ref[...], 1e-12)
    scores = lax.dot_general(
        qn_ref[...], mn, (((1,), (1,)), ((), ())),
        preferred_element_type=jnp.float32,
    )
    bmax = jnp.max(scores, axis=1, keepdims=True)

    eqf = jnp.where(scores == bmax, 1.0, 0.0)
    coli1 = lax.broadcasted_iota(jnp.int32, (BK, 1), 0)
    col_hi = (coli1 // 256).astype(jnp.float32)
    col_lo = (coli1 % 256).astype(jnp.float32)
    ones = jnp.ones((BK, 1), jnp.float32)
    aux = lax.dot_general(
        eqf, jnp.concatenate([col_hi, col_lo, ones], axis=1),
        (((1,), (0,)), ((), ())),
        preferred_element_type=jnp.float32,
    )
    tmp_ref[...] = (256 * aux[:, 0:1] + aux[:, 1:2]).astype(jnp.int32)

    @pl.when(jnp.max(aux[:, 2:3]) > 1.5)
    def _():
        coli = lax.broadcasted_iota(jnp.int32, scores.shape, 1)
        tmp_ref[...] = jnp.min(
            jnp.where(scores == bmax, coli, BK), axis=1, keepdims=True)

    bidx = tmp_ref[...] + k * BK

    @pl.when(k == 0)
    def _():
        score_ref[...] = bmax
        idx_ref[...] = bidx

    @pl.when(k > 0)
    def _():
        prev = score_ref[...]
        better = bmax > prev
        score_ref[...] = jnp.where(better, bmax, prev)
        idx_ref[...] = jnp.where(better, bidx, idx_ref[...])


def _build_topk(Q, D, K, BK, interpret=False):
    assert K % BK == 0 and BK % 8 == 0
    return pl.pallas_call(
        functools.partial(_topk_body, K, BK),
        grid=(K // BK,),
        in_specs=[
            pl.BlockSpec((Q, D), lambda k: (0, 0)),
            pl.BlockSpec((Q, 1), lambda k: (0, 0)),
            pl.BlockSpec((BK, D), lambda k: (k, 0)),
            pl.BlockSpec((BK, 1), lambda k: (k, 0)),
        ],
        out_specs=[
            pl.BlockSpec((Q, 1), lambda k: (0, 0)),
            pl.BlockSpec((Q, 1), lambda k: (0, 0)),
        ],
        out_shape=[
            jax.ShapeDtypeStruct((Q, 1), jnp.float32),
            jax.ShapeDtypeStruct((Q, 1), jnp.int32),
        ],
        scratch_shapes=[
            pltpu.VMEM((Q, D), jnp.float32),
            pltpu.VMEM((Q, 1), jnp.int32),
        ],
        interpret=interpret,
    )


def _build_sc_gather(Q, D):
    assert Q % (8 * _NUM_SC_WORKERS) == 0
    bpw = Q // _NUM_SC_WORKERS
    mesh = plsc.VectorSubcoreMesh(core_axis_name="c", subcore_axis_name="s")

    @functools.partial(
        pl.kernel,
        mesh=mesh,
        out_type=jax.ShapeDtypeStruct((Q, D), jnp.float32),
        scratch_types=[
            pltpu.VMEM((bpw,), jnp.int32),
            pltpu.VMEM((bpw, D), jnp.float32),
            pltpu.SemaphoreType.DMA,
        ],
    )
    def gather_kernel(table_hbm, idx_hbm, out_hbm, idx_v, rows_v, sem):
        wid = lax.axis_index("s") * _NUM_SC_CORES + lax.axis_index("c")
        base = wid * bpw
        pltpu.sync_copy(idx_hbm.at[pl.ds(base, bpw)], idx_v)
        pltpu.async_copy(table_hbm.at[idx_v], rows_v, sem).wait()
        pltpu.sync_copy(rows_v, out_hbm.at[pl.ds(base, bpw)])

    return gather_kernel


def kernel(x, memory_states):
    Q, D = x.shape
    K = memory_states.shape[0]
    BK = 2000
    xn = jnp.linalg.norm(x, ord=2, axis=1, keepdims=True)
    mnorm = jnp.linalg.norm(memory_states, ord=2, axis=1, keepdims=True)
    best_score, best_idx = _build_topk(Q, D, K, BK)(x, xn, memory_states, mnorm)
    gather = _build_sc_gather(Q, D)
    retrieved = gather(memory_states, best_idx.reshape(Q))
    return retrieved.reshape(Q, 1, D), best_score


# speedup vs baseline: 1.5239x; 1.5239x over previous
import functools
import jax
import jax.numpy as jnp
from jax import lax
from jax.experimental import pallas as pl
from jax.experimental.pallas import tpu as pltpu
from jax.experimental.pallas import tpu_sc as plsc

_NUM_SC_CORES = 2
_NUM_SC_SUBCORES = 16
_NUM_SC_WORKERS = _NUM_SC_CORES * _NUM_SC_SUBCORES


def _topk_body(K, BK, x_ref, xn_ref, mem_ref, mnorm_ref,
               score_ref, idx_ref, qn_ref):
    k = pl.program_id(0)

    @pl.when(k == 0)
    def _():
        qn_ref[...] = x_ref[...] / jnp.maximum(xn_ref[...], 1e-12)

    mn = mem_ref[...] / jnp.maximum(mnorm_ref[...], 1e-12)
    scores = lax.dot_general(
        qn_ref[...], mn, (((1,), (1,)), ((), ())),
        preferred_element_type=jnp.float32,
    )
    bmax = jnp.max(scores, axis=1, keepdims=True)

    col = lax.broadcasted_iota(jnp.int32, (1, BK), 1)
    bidx = jnp.min(
        jnp.where(scores == bmax, col, BK), axis=1, keepdims=True
    ) + k * BK

    @pl.when(k == 0)
    def _():
        score_ref[...] = bmax
        idx_ref[...] = bidx

    @pl.when(k > 0)
    def _():
        prev = score_ref[...]
        better = bmax > prev
        score_ref[...] = jnp.where(better, bmax, prev)
        idx_ref[...] = jnp.where(better, bidx, idx_ref[...])


def _build_topk(Q, D, K, BK, interpret=False):
    assert K % BK == 0 and BK % 8 == 0
    return pl.pallas_call(
        functools.partial(_topk_body, K, BK),
        grid=(K // BK,),
        in_specs=[
            pl.BlockSpec((Q, D), lambda k: (0, 0)),
            pl.BlockSpec((Q, 1), lambda k: (0, 0)),
            pl.BlockSpec((BK, D), lambda k: (k, 0)),
            pl.BlockSpec((BK, 1), lambda k: (k, 0)),
        ],
        out_specs=[
            pl.BlockSpec((Q, 1), lambda k: (0, 0)),
            pl.BlockSpec((Q, 1), lambda k: (0, 0)),
        ],
        out_shape=[
            jax.ShapeDtypeStruct((Q, 1), jnp.float32),
            jax.ShapeDtypeStruct((Q, 1), jnp.int32),
        ],
        scratch_shapes=[
            pltpu.VMEM((Q, D), jnp.float32),
        ],
        interpret=interpret,
    )


def _build_sc_gather(Q, D):
    assert Q % (8 * _NUM_SC_WORKERS) == 0
    bpw = Q // _NUM_SC_WORKERS
    mesh = plsc.VectorSubcoreMesh(core_axis_name="c", subcore_axis_name="s")

    @functools.partial(
        pl.kernel,
        mesh=mesh,
        out_type=jax.ShapeDtypeStruct((Q, D), jnp.float32),
        scratch_types=[
            pltpu.VMEM((bpw,), jnp.int32),
            pltpu.VMEM((bpw, D), jnp.float32),
            pltpu.SemaphoreType.DMA,
        ],
    )
    def gather_kernel(table_hbm, idx_hbm, out_hbm, idx_v, rows_v, sem):
        wid = lax.axis_index("s") * _NUM_SC_CORES + lax.axis_index("c")
        base = wid * bpw
        pltpu.sync_copy(idx_hbm.at[pl.ds(base, bpw)], idx_v)
        pltpu.async_copy(table_hbm.at[idx_v], rows_v, sem).wait()
        pltpu.sync_copy(rows_v, out_hbm.at[pl.ds(base, bpw)])

    return gather_kernel


def kernel(x, memory_states):
    Q, D = x.shape
    K = memory_states.shape[0]
    BK = 5000
    xn = jnp.linalg.norm(x, ord=2, axis=1, keepdims=True)
    mnorm = jnp.linalg.norm(memory_states, ord=2, axis=1, keepdims=True)
    best_score, best_idx = _build_topk(Q, D, K, BK)(x, xn, memory_states, mnorm)
    gather = _build_sc_gather(Q, D)
    retrieved = gather(memory_states, best_idx.reshape(Q))
    return retrieved.reshape(Q, 1, D), best_score


# speedup vs baseline: 1.8178x; 1.1929x over previous
import functools
import jax
import jax.numpy as jnp
from jax import lax
from jax.experimental import pallas as pl
from jax.experimental.pallas import tpu as pltpu
from jax.experimental.pallas import tpu_sc as plsc

_NUM_SC_CORES = 2
_NUM_SC_SUBCORES = 16
_NUM_SC_WORKERS = _NUM_SC_CORES * _NUM_SC_SUBCORES


def _topk_body(K, BK, x_ref, xn_ref, mem_ref, mnorm_ref,
               score_ref, idx_ref, qn_ref):
    k = pl.program_id(0)

    @pl.when(k == 0)
    def _():
        qn_ref[...] = x_ref[...] / jnp.maximum(xn_ref[...], 1e-12)

    nb = jnp.transpose(mnorm_ref[...].reshape(1, BK), (1, 0))
    mn = mem_ref[...] / jnp.maximum(nb, 1e-12)
    scores = lax.dot_general(
        qn_ref[...], mn, (((1,), (1,)), ((), ())),
        preferred_element_type=jnp.float32,
    )
    bmax = jnp.max(scores, axis=1, keepdims=True)

    col = lax.broadcasted_iota(jnp.int32, (1, BK), 1)
    bidx = jnp.min(
        jnp.where(scores == bmax, col, BK), axis=1, keepdims=True
    ) + k * BK

    @pl.when(k == 0)
    def _():
        score_ref[...] = bmax
        idx_ref[...] = bidx

    @pl.when(k > 0)
    def _():
        prev = score_ref[...]
        better = bmax > prev
        score_ref[...] = jnp.where(better, bmax, prev)
        idx_ref[...] = jnp.where(better, bidx, idx_ref[...])


def _build_topk(Q, D, K, BK, interpret=False):
    assert K % BK == 0 and BK % 8 == 0
    return pl.pallas_call(
        functools.partial(_topk_body, K, BK),
        grid=(K // BK,),
        in_specs=[
            pl.BlockSpec((Q, D), lambda k: (0, 0)),
            pl.BlockSpec((Q, 1), lambda k: (0, 0)),
            pl.BlockSpec((BK, D), lambda k: (k, 0)),
            pl.BlockSpec((1, 1, BK), lambda k: (k, 0, 0)),
        ],
        out_specs=[
            pl.BlockSpec((Q, 1), lambda k: (0, 0)),
            pl.BlockSpec((Q, 1), lambda k: (0, 0)),
        ],
        out_shape=[
            jax.ShapeDtypeStruct((Q, 1), jnp.float32),
            jax.ShapeDtypeStruct((Q, 1), jnp.int32),
        ],
        scratch_shapes=[
            pltpu.VMEM((Q, D), jnp.float32),
        ],
        interpret=interpret,
    )


def _build_sc_gather(Q, D):
    assert Q % (8 * _NUM_SC_WORKERS) == 0
    bpw = Q // _NUM_SC_WORKERS
    mesh = plsc.VectorSubcoreMesh(core_axis_name="c", subcore_axis_name="s")

    @functools.partial(
        pl.kernel,
        mesh=mesh,
        out_type=jax.ShapeDtypeStruct((Q, D), jnp.float32),
        scratch_types=[
            pltpu.VMEM((bpw,), jnp.int32),
            pltpu.VMEM((bpw, D), jnp.float32),
            pltpu.SemaphoreType.DMA,
        ],
    )
    def gather_kernel(table_hbm, idx_hbm, out_hbm, idx_v, rows_v, sem):
        wid = lax.axis_index("s") * _NUM_SC_CORES + lax.axis_index("c")
        base = wid * bpw
        pltpu.sync_copy(idx_hbm.at[pl.ds(base, bpw)], idx_v)
        pltpu.async_copy(table_hbm.at[idx_v], rows_v, sem).wait()
        pltpu.sync_copy(rows_v, out_hbm.at[pl.ds(base, bpw)])

    return gather_kernel


def kernel(x, memory_states):
    Q, D = x.shape
    K = memory_states.shape[0]
    BK = 5000
    xn = jnp.linalg.norm(x, ord=2, axis=1, keepdims=True)
    mnorm = jnp.linalg.norm(memory_states, ord=2, axis=1, keepdims=True)
    mnorm_row = mnorm.reshape(K // BK, 1, BK)
    best_score, best_idx = _build_topk(Q, D, K, BK)(x, xn, memory_states, mnorm_row)
    gather = _build_sc_gather(Q, D)
    retrieved = gather(memory_states, best_idx.reshape(Q))
    return retrieved.reshape(Q, 1, D), best_score


# speedup vs baseline: 1.8884x; 1.0388x over previous
import functools
import jax
import jax.numpy as jnp
from jax import lax
from jax.experimental import pallas as pl
from jax.experimental.pallas import tpu as pltpu
from jax.experimental.pallas import tpu_sc as plsc

_NUM_SC_CORES = 2
_NUM_SC_SUBCORES = 16
_NUM_SC_WORKERS = _NUM_SC_CORES * _NUM_SC_SUBCORES


def _topk_body(K, BK, x_ref, xn_ref, mem_ref, mnorm_ref,
               score_ref, idx_ref, qn_ref):
    k = pl.program_id(0)

    @pl.when(k == 0)
    def _():
        qn_ref[...] = x_ref[...] / jnp.maximum(xn_ref[...], 1e-12)

    nb = jnp.transpose(mnorm_ref[...].reshape(1, BK), (1, 0))
    mn = mem_ref[...] / jnp.maximum(nb, 1e-12)
    scores = lax.dot_general(
        qn_ref[...], mn, (((1,), (1,)), ((), ())),
        preferred_element_type=jnp.float32,
    )
    bmax = jnp.max(scores, axis=1, keepdims=True)

    col = lax.broadcasted_iota(jnp.int32, (1, BK), 1)
    bidx = jnp.min(
        jnp.where(scores == bmax, col, BK), axis=1, keepdims=True
    ) + k * BK

    @pl.when(k == 0)
    def _():
        score_ref[...] = bmax
        idx_ref[...] = bidx

    @pl.when(k > 0)
    def _():
        prev = score_ref[...]
        better = bmax > prev
        score_ref[...] = jnp.where(better, bmax, prev)
        idx_ref[...] = jnp.where(better, bidx, idx_ref[...])


def _build_topk(Q, D, K, BK, interpret=False):
    assert K % BK == 0 and BK % 8 == 0
    return pl.pallas_call(
        functools.partial(_topk_body, K, BK),
        grid=(K // BK,),
        in_specs=[
            pl.BlockSpec((Q, D), lambda k: (0, 0)),
            pl.BlockSpec((Q, 1), lambda k: (0, 0)),
            pl.BlockSpec((BK, D), lambda k: (k, 0)),
            pl.BlockSpec((1, 1, BK), lambda k: (k, 0, 0)),
        ],
        out_specs=[
            pl.BlockSpec((Q, 1), lambda k: (0, 0)),
            pl.BlockSpec((Q, 1), lambda k: (0, 0)),
        ],
        out_shape=[
            jax.ShapeDtypeStruct((Q, 1), jnp.float32),
            jax.ShapeDtypeStruct((Q, 1), jnp.int32),
        ],
        scratch_shapes=[
            pltpu.VMEM((Q, D), jnp.float32),
        ],
        compiler_params=pltpu.CompilerParams(
            vmem_limit_bytes=120 * 1024 * 1024),
        interpret=interpret,
    )


def _build_sc_gather(Q, D):
    assert Q % (8 * _NUM_SC_WORKERS) == 0
    bpw = Q // _NUM_SC_WORKERS
    mesh = plsc.VectorSubcoreMesh(core_axis_name="c", subcore_axis_name="s")

    @functools.partial(
        pl.kernel,
        mesh=mesh,
        out_type=jax.ShapeDtypeStruct((Q, D), jnp.float32),
        scratch_types=[
            pltpu.VMEM((bpw,), jnp.int32),
            pltpu.VMEM((bpw, D), jnp.float32),
            pltpu.SemaphoreType.DMA,
        ],
    )
    def gather_kernel(table_hbm, idx_hbm, out_hbm, idx_v, rows_v, sem):
        wid = lax.axis_index("s") * _NUM_SC_CORES + lax.axis_index("c")
        base = wid * bpw
        pltpu.sync_copy(idx_hbm.at[pl.ds(base, bpw)], idx_v)
        pltpu.async_copy(table_hbm.at[idx_v], rows_v, sem).wait()
        pltpu.sync_copy(rows_v, out_hbm.at[pl.ds(base, bpw)])

    return gather_kernel


def kernel(x, memory_states):
    Q, D = x.shape
    K = memory_states.shape[0]
    BK = 10000
    xn = jnp.linalg.norm(x, ord=2, axis=1, keepdims=True)
    mnorm = jnp.linalg.norm(memory_states, ord=2, axis=1, keepdims=True)
    mnorm_row = mnorm.reshape(K // BK, 1, BK)
    best_score, best_idx = _build_topk(Q, D, K, BK)(x, xn, memory_states, mnorm_row)
    gather = _build_sc_gather(Q, D)
    retrieved = gather(memory_states, best_idx.reshape(Q))
    return retrieved.reshape(Q, 1, D), best_score
